# baseline (device time: 778235 ns/iter reference)
import jax
import jax.numpy as jnp
from jax import lax
from jax.experimental import pallas as pl
from jax.experimental.pallas import tpu as pltpu

N_DEV = 32
N_TOK = 512
D = 256
H = 512
E_LOC = 4
N_EXP = 128
XCOL = 0
ACOL = D
RCOL = D + H
WIDTH = 896


def kernel(x, router_W, route_idx, expert_W, shared_W):
    def body(x_ref, router_ref, route_ref, expert_ref, shared_ref,
             out_ref, data, send_sems, recv_sems, credit_sem):
        my = lax.axis_index("i")
        left = lax.rem(my + N_DEV - 1, N_DEV)
        right = lax.rem(my + 1, N_DEV)

        barrier = pltpu.get_barrier_semaphore()
        for nbr in (left, right):
            pl.semaphore_signal(barrier, inc=1, device_id=(nbr,),
                                device_id_type=pl.DeviceIdType.MESH)
        pl.semaphore_wait(barrier, 2)

        xv = x_ref[:, :]
        scores = jnp.dot(xv, router_ref[:, :],
                         preferred_element_type=jnp.float32)
        scores = scores - jnp.max(scores, axis=1, keepdims=True)
        p = jnp.exp(scores)
        p = p / jnp.sum(p, axis=1, keepdims=True)
        eids = lax.broadcasted_iota(jnp.int32, (N_TOK, N_EXP), 1)
        w = jnp.sum(jnp.where(eids == route_ref[:, :], p, 0.0),
                    axis=1, keepdims=True)
        data[0, :, XCOL:XCOL + D] = xv * w
        data[0, :, ACOL:ACOL + H] = jnp.dot(
            xv, shared_ref[:, :], preferred_element_type=jnp.float32)
        data[0, :, RCOL:RCOL + 1] = route_ref[:, :].astype(jnp.float32)

        ew = expert_ref[:, :, :].reshape(E_LOC * D, H)
        base = my * E_LOC

        for h in range(N_DEV):
            s = h % 2
            if h > 0:
                recv = pltpu.make_async_remote_copy(
                    src_ref=data.at[s], dst_ref=data.at[s],
                    send_sem=send_sems.at[s], recv_sem=recv_sems.at[s],
                    device_id=(left,), device_id_type=pl.DeviceIdType.MESH)
                recv.wait_recv()

            xb = data[s, :, XCOL:XCOL + D]
            rt = data[s, :, RCOL:RCOL + 1]
            parts = [
                jnp.where(rt == (base + j).astype(jnp.float32), xb, 0.0)
                for j in range(E_LOC)
            ]
            xs = jnp.concatenate(parts, axis=1)
            contrib = jnp.dot(xs, ew, preferred_element_type=jnp.float32)
            data[s, :, ACOL:ACOL + H] = data[s, :, ACOL:ACOL + H] + contrib

            if h > 0:
                pl.semaphore_wait(credit_sem, 1)
            send = pltpu.make_async_remote_copy(
                src_ref=data.at[s], dst_ref=data.at[1 - s],
                send_sem=send_sems.at[s], recv_sem=recv_sems.at[1 - s],
                device_id=(right,), device_id_type=pl.DeviceIdType.MESH)
            send.start()
            send.wait_send()
            if h < N_DEV - 1:
                pl.semaphore_signal(credit_sem, inc=1, device_id=(left,),
                                    device_id_type=pl.DeviceIdType.MESH)

        fin = pltpu.make_async_remote_copy(
            src_ref=data.at[0], dst_ref=data.at[0],
            send_sem=send_sems.at[0], recv_sem=recv_sems.at[0],
            device_id=(left,), device_id_type=pl.DeviceIdType.MESH)
        fin.wait_recv()
        out_ref[:, :] = data[0, :, ACOL:ACOL + H]

    return pl.pallas_call(
        body,
        out_shape=jax.ShapeDtypeStruct((N_TOK, H), jnp.float32),
        in_specs=[pl.BlockSpec(memory_space=pltpu.VMEM)] * 5,
        out_specs=pl.BlockSpec(memory_space=pltpu.VMEM),
        scratch_shapes=[
            pltpu.VMEM((2, N_TOK, WIDTH), jnp.float32),
            pltpu.SemaphoreType.DMA((2,)),
            pltpu.SemaphoreType.DMA((2,)),
            pltpu.SemaphoreType.REGULAR,
        ],
        compiler_params=pltpu.CompilerParams(collective_id=0),
    )(x, router_W, route_idx, expert_W, shared_W)
